# baseline (speedup 1.0000x reference)
import functools

import jax
import jax.numpy as jnp
from jax import lax
from jax.experimental import pallas as pl
from jax.experimental.pallas import tpu as pltpu
from jax.experimental.pallas import tpu_sc as plsc

N = 10000
E = 320000
D = 128
B = 64
EPS_GIN = 1.0


_SC_K = 80
_SC_NBUF = 2


def _make_sc_scatter():
    info = plsc.get_sparse_core_info()
    NC, NS = info.num_cores, info.num_subcores
    NW = NC * NS
    EPW = E // NW
    K = _SC_K
    CHUNKS = EPW // K
    NBUF = _SC_NBUF
    OUTER = (CHUNKS + NBUF - 1) // NBUF
    RPT = (N // NS) // 8 * 8
    TAIL = N - NS * RPT

    mesh = plsc.VectorSubcoreMesh(core_axis_name="c", subcore_axis_name="s")

    @functools.partial(
        pl.kernel,
        out_type=jax.ShapeDtypeStruct((NC, N, D), jnp.float32),
        mesh=mesh,
        scratch_types=[
            pltpu.VMEM((EPW,), jnp.int32),
            pltpu.VMEM((CHUNKS, K), jnp.int32),
            [pltpu.VMEM((K, D), jnp.float32) for _ in range(NBUF)],
            [pltpu.SemaphoreType.DMA for _ in range(NBUF)],
            pltpu.SemaphoreType.DMA,
            pltpu.SemaphoreType.DMA,
            pltpu.VMEM_SHARED((N, D), jnp.float32),
        ],
    )
    def sc_scatter(x_hbm, src_hbm, dst_hbm, zero_hbm, out_hbm,
                   src_v, dst_v, rows, gsems, zsem, isem, acc_sh):
        c = lax.axis_index("c")
        s = lax.axis_index("s")
        wid = s * NC + c

        zero_cp = pltpu.make_async_copy(zero_hbm,
                                        acc_sh.at[pl.ds(s * RPT, RPT)], zsem)
        zero_cp.start()
        src_cp = pltpu.make_async_copy(src_hbm.at[wid], src_v, isem)
        src_cp.start()
        dst_cp = pltpu.make_async_copy(dst_hbm.at[wid], dst_v, isem)
        dst_cp.start()

        @pl.when(s == NS - 1)
        def _zero_tail():
            pltpu.sync_copy(zero_hbm.at[pl.ds(0, TAIL)],
                            acc_sh.at[pl.ds(NS * RPT, TAIL)])

        src_cp.wait()
        dst_cp.wait()
        zero_cp.wait()
        plsc.subcore_barrier()

        for b in range(NBUF):
            pltpu.async_copy(x_hbm.at[src_v.at[pl.ds(b * K, K)]], rows[b],
                             gsems[b])

        @pl.loop(0, (CHUNKS - NBUF - 1) // NBUF)
        def _outer(o):
            for b in range(NBUF):
                i = o * NBUF + b
                pltpu.make_async_copy(x_hbm.at[pl.ds(0, K)], rows[b],
                                      gsems[b]).wait()
                pltpu.sync_copy(rows[b], acc_sh.at[dst_v.at[i]], add=True)
                pltpu.async_copy(
                    x_hbm.at[src_v.at[pl.ds((i + NBUF) * K, K)]],
                    rows[b], gsems[b])

        for i in range(((CHUNKS - NBUF - 1) // NBUF) * NBUF, CHUNKS):
            b = i % NBUF
            pltpu.make_async_copy(x_hbm.at[pl.ds(0, K)], rows[b],
                                  gsems[b]).wait()
            pltpu.sync_copy(rows[b], acc_sh.at[dst_v.at[i]], add=True)
            if i + NBUF < CHUNKS:
                pltpu.async_copy(
                    x_hbm.at[src_v.at[pl.ds((i + NBUF) * K, K)]],
                    rows[b], gsems[b])

        plsc.subcore_barrier()
        pltpu.sync_copy(acc_sh.at[pl.ds(s * RPT, RPT)],
                        out_hbm.at[c].at[pl.ds(s * RPT, RPT)])

        @pl.when(s == NS - 1)
        def _write_tail():
            pltpu.sync_copy(acc_sh.at[pl.ds(NS * RPT, TAIL)],
                            out_hbm.at[c].at[pl.ds(NS * RPT, TAIL)])

    return sc_scatter


_sc_scatter = _make_sc_scatter()


def _tc_body(x_ref, parts_ref, batch_ref, W1_ref, b1_ref, W2_ref,
             b2_ref, ln1w_ref, ln1b_ref, ln2w_ref, ln2b_ref,
             out_x_ref, out_g_ref):
    x = x_ref[...]
    gids = lax.broadcasted_iota(jnp.int32, (N, B), 1)
    M = (batch_ref[...] == gids).astype(jnp.float32)
    ones_col = jnp.ones((N, 1), jnp.float32)
    deg = lax.dot_general(M, ones_col, (((0,), (0,)), ((), ())),
                          preferred_element_type=jnp.float32)
    deg = jnp.maximum(deg, 1.0)
    inv_norm = 1.0 / (deg * float(D))

    def layer_norm(h, w, bias):
        rs = jnp.sum(h, axis=1, keepdims=True)
        seg = lax.dot_general(M, rs, (((0,), (0,)), ((), ())),
                              preferred_element_type=jnp.float32)
        mean_g = seg * inv_norm
        mean_n = jnp.dot(M, mean_g, preferred_element_type=jnp.float32)
        hc = h - mean_n
        rs2 = jnp.sum(hc * hc, axis=1, keepdims=True)
        var_g = lax.dot_general(M, rs2, (((0,), (0,)), ((), ())),
                                preferred_element_type=jnp.float32) * inv_norm
        inv_g = lax.rsqrt(var_g + 1e-5)
        inv_n = jnp.dot(M, inv_g, preferred_element_type=jnp.float32)
        return hc * inv_n * w + bias

    hin = (1.0 + EPS_GIN) * x + parts_ref[0] + parts_ref[1]
    h = jnp.dot(hin, W1_ref[...], preferred_element_type=jnp.float32) + b1_ref[...]
    h = layer_norm(h, ln1w_ref[0, 0], ln1b_ref[0, 0])
    x1 = x + jnp.maximum(h, 0.0)

    h2 = jnp.dot(x1, W2_ref[...], preferred_element_type=jnp.float32) + b2_ref[...]
    h2 = layer_norm(h2, ln2w_ref[0, 0], ln2b_ref[0, 0])
    x2 = jnp.maximum(h2, 0.0)
    out_x_ref[...] = x2

    pool = lax.dot_general(M, x2, (((0,), (0,)), ((), ())),
                           preferred_element_type=jnp.float32)
    out_g_ref[...] = pool / deg


_tc_fused = pl.pallas_call(
    _tc_body,
    out_shape=[
        jax.ShapeDtypeStruct((N, D), jnp.float32),
        jax.ShapeDtypeStruct((B, D), jnp.float32),
    ],
)


def kernel(x, edge_index, batch, W1, b1, ln1_w, ln1_b, W2, b2, ln2_w, ln2_b):
    nw = 32
    epw = E // nw
    chunks = epw // _SC_K
    src = edge_index[0].reshape(nw, epw)
    dst = edge_index[1].reshape(nw, chunks, _SC_K)
    zero_block = jnp.zeros((624, D), jnp.float32)
    parts = _sc_scatter(x, src, dst, zero_block)
    out_x, out_g = _tc_fused(
        x, parts, batch.reshape(N, 1),
        W1, b1.reshape(1, D), W2, b2.reshape(1, D),
        ln1_w.reshape(1, 1), ln1_b.reshape(1, 1),
        ln2_w.reshape(1, 1), ln2_b.reshape(1, 1),
    )
    return (out_x, out_g)


# speedup vs baseline: 1.0488x; 1.0488x over previous
import functools

import jax
import jax.numpy as jnp
from jax import lax
from jax.experimental import pallas as pl
from jax.experimental.pallas import tpu as pltpu
from jax.experimental.pallas import tpu_sc as plsc

N = 10000
E = 320000
D = 128
B = 64
EPS_GIN = 1.0


_SC_K = 128
_SC_NBUF = 2


def _make_sc_scatter():
    info = plsc.get_sparse_core_info()
    NC, NS = info.num_cores, info.num_subcores
    NW = NC * NS
    EPW = E // NW
    K = _SC_K
    CHUNKS = EPW // K
    TAIL_E = EPW - CHUNKS * K
    NBUF = _SC_NBUF
    RPT = (N // NS) // 8 * 8
    TAIL = N - NS * RPT

    mesh = plsc.VectorSubcoreMesh(core_axis_name="c", subcore_axis_name="s")

    @functools.partial(
        pl.kernel,
        out_type=jax.ShapeDtypeStruct((NC, N, D), jnp.float32),
        mesh=mesh,
        scratch_types=[
            pltpu.VMEM((EPW,), jnp.int32),
            [pltpu.VMEM((K,), jnp.int32) for _ in range(NBUF)],
            pltpu.VMEM((TAIL_E,), jnp.int32),
            [pltpu.VMEM((K, D), jnp.float32) for _ in range(NBUF)],
            [pltpu.SemaphoreType.DMA for _ in range(NBUF)],
            [pltpu.SemaphoreType.DMA for _ in range(NBUF)],
            pltpu.SemaphoreType.DMA,
            pltpu.SemaphoreType.DMA,
            pltpu.VMEM_SHARED((N, D), jnp.float32),
        ],
    )
    def sc_scatter(x_hbm, src_hbm, dst_hbm, zero_hbm, out_hbm,
                   src_v, dsts, dst_t, rows, gsems, dsems, zsem, isem, acc_sh):
        c = lax.axis_index("c")
        s = lax.axis_index("s")
        wid = s * NC + c

        zero_cp = pltpu.make_async_copy(zero_hbm,
                                        acc_sh.at[pl.ds(s * RPT, RPT)], zsem)
        zero_cp.start()
        src_cp = pltpu.make_async_copy(src_hbm.at[wid], src_v, isem)
        src_cp.start()

        @pl.when(s == NS - 1)
        def _zero_tail():
            pltpu.sync_copy(zero_hbm.at[pl.ds(0, TAIL)],
                            acc_sh.at[pl.ds(NS * RPT, TAIL)])

        src_cp.wait()
        zero_cp.wait()
        plsc.subcore_barrier()

        dst_row = dst_hbm.at[wid]

        for b in range(NBUF):
            pltpu.async_copy(dst_row.at[pl.ds(b * K, K)], dsts[b], dsems[b])
            pltpu.async_copy(x_hbm.at[src_v.at[pl.ds(b * K, K)]], rows[b],
                             gsems[b])

        def step(i, b, refill):
            pltpu.make_async_copy(x_hbm.at[pl.ds(0, K)], rows[b],
                                  gsems[b]).wait()
            pltpu.make_async_copy(dst_row.at[pl.ds(0, K)], dsts[b],
                                  dsems[b]).wait()
            pltpu.sync_copy(rows[b], acc_sh.at[dsts[b]], add=True)
            if refill:
                nxt = i + NBUF
                pltpu.async_copy(dst_row.at[pl.ds(nxt * K, K)], dsts[b],
                                 dsems[b])
                pltpu.async_copy(x_hbm.at[src_v.at[pl.ds(nxt * K, K)]],
                                 rows[b], gsems[b])

        MAIN = (CHUNKS - NBUF - 1) // NBUF

        @pl.loop(0, MAIN)
        def _outer(o):
            for b in range(NBUF):
                step(o * NBUF + b, b, True)

        for i in range(MAIN * NBUF, CHUNKS):
            step(i, i % NBUF, i + NBUF < CHUNKS)

        pltpu.async_copy(dst_row.at[pl.ds(CHUNKS * K, TAIL_E)], dst_t, isem)
        pltpu.async_copy(x_hbm.at[src_v.at[pl.ds(CHUNKS * K, TAIL_E)]],
                         rows[0].at[pl.ds(0, TAIL_E)], gsems[0])
        pltpu.make_async_copy(dst_row.at[pl.ds(0, TAIL_E)], dst_t,
                              isem).wait()
        pltpu.make_async_copy(x_hbm.at[pl.ds(0, TAIL_E)],
                              rows[0].at[pl.ds(0, TAIL_E)], gsems[0]).wait()
        pltpu.sync_copy(rows[0].at[pl.ds(0, TAIL_E)], acc_sh.at[dst_t],
                        add=True)

        plsc.subcore_barrier()
        pltpu.sync_copy(acc_sh.at[pl.ds(s * RPT, RPT)],
                        out_hbm.at[c].at[pl.ds(s * RPT, RPT)])

        @pl.when(s == NS - 1)
        def _write_tail():
            pltpu.sync_copy(acc_sh.at[pl.ds(NS * RPT, TAIL)],
                            out_hbm.at[c].at[pl.ds(NS * RPT, TAIL)])

    return sc_scatter


_sc_scatter = _make_sc_scatter()


def _tc_body(x_ref, parts_ref, batch_ref, W1_ref, b1_ref, W2_ref,
             b2_ref, ln1w_ref, ln1b_ref, ln2w_ref, ln2b_ref,
             out_x_ref, out_g_ref):
    x = x_ref[...]
    gids = lax.broadcasted_iota(jnp.int32, (N, B), 1)
    M = (batch_ref[...] == gids).astype(jnp.float32)
    ones_col = jnp.ones((N, 1), jnp.float32)
    deg = lax.dot_general(M, ones_col, (((0,), (0,)), ((), ())),
                          preferred_element_type=jnp.float32)
    deg = jnp.maximum(deg, 1.0)
    inv_norm = 1.0 / (deg * float(D))

    def layer_norm(h, w, bias):
        rs = jnp.sum(h, axis=1, keepdims=True)
        seg = lax.dot_general(M, rs, (((0,), (0,)), ((), ())),
                              preferred_element_type=jnp.float32)
        mean_g = seg * inv_norm
        mean_n = jnp.dot(M, mean_g, preferred_element_type=jnp.float32)
        hc = h - mean_n
        rs2 = jnp.sum(hc * hc, axis=1, keepdims=True)
        var_g = lax.dot_general(M, rs2, (((0,), (0,)), ((), ())),
                                preferred_element_type=jnp.float32) * inv_norm
        inv_g = lax.rsqrt(var_g + 1e-5)
        inv_n = jnp.dot(M, inv_g, preferred_element_type=jnp.float32)
        return hc * inv_n * w + bias

    hin = (1.0 + EPS_GIN) * x + parts_ref[0] + parts_ref[1]
    h = jnp.dot(hin, W1_ref[...], preferred_element_type=jnp.float32) + b1_ref[...]
    h = layer_norm(h, ln1w_ref[0, 0], ln1b_ref[0, 0])
    x1 = x + jnp.maximum(h, 0.0)

    h2 = jnp.dot(x1, W2_ref[...], preferred_element_type=jnp.float32) + b2_ref[...]
    h2 = layer_norm(h2, ln2w_ref[0, 0], ln2b_ref[0, 0])
    x2 = jnp.maximum(h2, 0.0)
    out_x_ref[...] = x2

    pool = lax.dot_general(M, x2, (((0,), (0,)), ((), ())),
                           preferred_element_type=jnp.float32)
    out_g_ref[...] = pool / deg


_tc_fused = pl.pallas_call(
    _tc_body,
    out_shape=[
        jax.ShapeDtypeStruct((N, D), jnp.float32),
        jax.ShapeDtypeStruct((B, D), jnp.float32),
    ],
)


def kernel(x, edge_index, batch, W1, b1, ln1_w, ln1_b, W2, b2, ln2_w, ln2_b):
    nw = 32
    epw = E // nw
    chunks = epw // _SC_K
    src = edge_index[0].reshape(nw, epw)
    dst = edge_index[1].reshape(nw, epw)
    zero_block = jnp.zeros((624, D), jnp.float32)
    parts = _sc_scatter(x, src, dst, zero_block)
    out_x, out_g = _tc_fused(
        x, parts, batch.reshape(N, 1),
        W1, b1.reshape(1, D), W2, b2.reshape(1, D),
        ln1_w.reshape(1, 1), ln1_b.reshape(1, 1),
        ln2_w.reshape(1, 1), ln2_b.reshape(1, 1),
    )
    return (out_x, out_g)
